# baseline (speedup 1.0000x reference)
import functools
import math

import jax
import jax.numpy as jnp
from jax import lax
from jax.experimental import pallas as pl
from jax.experimental.pallas import tpu as pltpu
from jax.experimental.pallas import tpu_sc as plsc

N = 10000
D = 128
ALPHA = 0.1
THETA = 0.5
LAYER = 4
BETA = math.log(THETA / (LAYER + 1) + 1.0)

NC = 2
NS = 16
NW = NC * NS
CHUNK = 128
ROWS_PER_SUB = 624
ROWS_LAST = N - (NS - 1) * ROWS_PER_SUB


def _sc_aggregate(x, ei, zeros_blk):
    E = ei.shape[1]
    assert E % CHUNK == 0
    nchunks = E // CHUNK
    full = nchunks // NW
    rem = nchunks % NW
    assert full % 3 == 0 and full >= 3
    mesh = plsc.VectorSubcoreMesh(core_axis_name="c", subcore_axis_name="s")

    @functools.partial(
        pl.kernel,
        out_type=jax.ShapeDtypeStruct((NC, N, D), jnp.float32),
        mesh=mesh,
        scratch_types=[
            pltpu.VMEM((4, CHUNK // 2), jnp.int32),
            pltpu.VMEM((4, CHUNK // 2), jnp.int32),
            pltpu.VMEM((4, CHUNK // 2), jnp.int32),
            pltpu.VMEM((CHUNK, D), jnp.float32),
            pltpu.VMEM((CHUNK, D), jnp.float32),
            pltpu.VMEM((CHUNK, D), jnp.float32),
            pltpu.VMEM_SHARED((N, D), jnp.float32),
            pltpu.SemaphoreType.DMA,
            pltpu.SemaphoreType.DMA,
            pltpu.SemaphoreType.DMA,
            pltpu.SemaphoreType.DMA,
            pltpu.SemaphoreType.DMA,
            pltpu.SemaphoreType.DMA,
            pltpu.SemaphoreType.DMA,
            pltpu.SemaphoreType.DMA,
            pltpu.SemaphoreType.DMA,
            pltpu.SemaphoreType.DMA,
        ],
    )
    def k(x_hbm, ei_hbm, z_hbm, out_hbm,
          idx0, idx1, idx2, rows0, rows1, rows2, acc,
          isem0, isem1, isem2, gsem0, gsem1, gsem2,
          ssem0, ssem1, ssem2, zsem):
        cid = lax.axis_index("c")
        sid = lax.axis_index("s")
        wid = sid * NC + cid
        tw = full + jnp.where(wid < rem, 1, 0)
        base = pl.multiple_of(sid * ROWS_PER_SUB, 8)
        last = sid == NS - 1

        ibufs = (idx0, idx1, idx2)
        isems = (isem0, isem1, isem2)
        rbufs = (rows0, rows1, rows2)
        gsems = (gsem0, gsem1, gsem2)

        H = CHUNK // 2

        def issue_idx(ch, ibuf, isem):
            off = pl.multiple_of((ch * NW + wid) * CHUNK, CHUNK)
            for r in (0, 1):
                pltpu.async_copy(
                    ei_hbm.at[0, pl.ds(off + r * H, H)], ibuf.at[r], isem)
            for r in (0, 1):
                pltpu.async_copy(
                    ei_hbm.at[1, pl.ds(off + r * H, H)], ibuf.at[2 + r], isem)

        def wait_idx(ch, ibuf, isem):
            off = pl.multiple_of((ch * NW + wid) * CHUNK, CHUNK)
            for r in (0, 1):
                pltpu.make_async_copy(
                    ei_hbm.at[0, pl.ds(off + r * H, H)], ibuf.at[r], isem).wait()
            for r in (0, 1):
                pltpu.make_async_copy(
                    ei_hbm.at[1, pl.ds(off + r * H, H)],
                    ibuf.at[2 + r], isem).wait()

        def issue_gather(ibuf, rows, gsem):
            for r in (0, 1):
                pltpu.async_copy(
                    x_hbm.at[ibuf.at[r]], rows.at[pl.ds(r * H, H)], gsem)

        def wait_gather(ibuf, rows, gsem):
            for r in (0, 1):
                pltpu.make_async_copy(
                    x_hbm.at[ibuf.at[r]], rows.at[pl.ds(r * H, H)], gsem).wait()

        def issue_scatter(rows, ibuf, ssem):
            for r in (0, 1):
                pltpu.async_copy(
                    rows.at[pl.ds(r * H, H)], acc.at[ibuf.at[2 + r]],
                    ssem, add=True)

        def wait_scatter(rows, ibuf, ssem):
            for r in (0, 1):
                pltpu.make_async_copy(
                    rows.at[pl.ds(r * H, H)], acc.at[ibuf.at[2 + r]],
                    ssem).wait()

        ssems = (ssem0, ssem1, ssem2)

        issue_idx(0, idx0, isem0)
        issue_idx(1, idx1, isem1)
        issue_idx(2, idx2, isem2)
        @pl.when(last)
        def _():
            pltpu.async_copy(z_hbm, acc.at[pl.ds(base, ROWS_LAST)], zsem)

        @pl.when(jnp.logical_not(last))
        def _():
            pltpu.async_copy(z_hbm.at[pl.ds(0, ROWS_PER_SUB)],
                             acc.at[pl.ds(base, ROWS_PER_SUB)], zsem)
        wait_idx(0, idx0, isem0)
        issue_gather(idx0, rows0, gsem0)
        wait_idx(1, idx1, isem1)
        issue_gather(idx1, rows1, gsem1)

        @pl.when(last)
        def _():
            pltpu.make_async_copy(
                z_hbm, acc.at[pl.ds(base, ROWS_LAST)], zsem).wait()

        @pl.when(jnp.logical_not(last))
        def _():
            pltpu.make_async_copy(z_hbm.at[pl.ds(0, ROWS_PER_SUB)],
                                  acc.at[pl.ds(base, ROWS_PER_SUB)],
                                  zsem).wait()
        plsc.subcore_barrier()

        def body(i, carry):
            for b in (0, 1, 2):
                ch = 3 * i + b
                ibuf, isem = ibufs[b], isems[b]
                rows, gsem, ssem = rbufs[b], gsems[b], ssems[b]
                b2 = (b + 2) % 3
                i2buf, i2sem = ibufs[b2], isems[b2]
                r2buf, g2sem, s2sem = rbufs[b2], gsems[b2], ssems[b2]

                wait_gather(ibuf, rows, gsem)

                @pl.when(ch + 2 < tw)
                def _():
                    wait_idx(ch + 2, i2buf, i2sem)

                    @pl.when(ch >= 1)
                    def _():
                        wait_scatter(r2buf, i2buf, s2sem)

                    issue_gather(i2buf, r2buf, g2sem)

                issue_scatter(rows, ibuf, ssem)

                @pl.when(ch + 3 < tw)
                def _():
                    issue_idx(ch + 3, ibuf, isem)
            return carry

        lax.fori_loop(0, full // 3, body, 0)

        @pl.when(tw > full)
        def _():
            wait_gather(idx0, rows0, gsem0)
            issue_scatter(rows0, idx0, ssem0)

        for b in (0, 1, 2):
            wait_scatter(rbufs[b], ibufs[b], ssems[b])

        plsc.subcore_barrier()

        @pl.when(last)
        def _():
            pltpu.sync_copy(acc.at[pl.ds(base, ROWS_LAST)],
                            out_hbm.at[cid, pl.ds(base, ROWS_LAST)])

        @pl.when(jnp.logical_not(last))
        def _():
            pltpu.sync_copy(acc.at[pl.ds(base, ROWS_PER_SUB)],
                            out_hbm.at[cid, pl.ds(base, ROWS_PER_SUB)])

    return k(x, ei, zeros_blk)


_BLK = 5000


def _tc_combine(partial, x_0, W):
    def body(p_ref, x0_ref, w_ref, out_ref):
        t = (1.0 - ALPHA) * (p_ref[0] + p_ref[1]) + ALPHA * x0_ref[...]
        out_ref[...] = (1.0 - BETA) * t + BETA * jnp.dot(
            t, w_ref[...], preferred_element_type=jnp.float32)

    bs = pl.BlockSpec((_BLK, D), lambda i: (i, 0))
    return pl.pallas_call(
        body,
        grid=(N // _BLK,),
        in_specs=[
            pl.BlockSpec((NC, _BLK, D), lambda i: (0, i, 0)),
            bs,
            pl.BlockSpec((D, D), lambda i: (0, 0)),
        ],
        out_specs=bs,
        out_shape=jax.ShapeDtypeStruct((N, D), jnp.float32),
    )(partial, x_0, W)


@jax.jit
def _impl(x, x_0, edge_index, W):
    ei = edge_index.astype(jnp.int32)
    zeros_blk = jnp.zeros((ROWS_LAST, D), jnp.float32)
    partial = _sc_aggregate(x, ei, zeros_blk)
    return _tc_combine(partial, x_0, W)


def kernel(x, x_0, edge_index, W):
    return _impl(x, x_0, edge_index, W)


# speedup vs baseline: 1.0209x; 1.0209x over previous
import functools
import math

import jax
import jax.numpy as jnp
from jax import lax
from jax.experimental import pallas as pl
from jax.experimental.pallas import tpu as pltpu
from jax.experimental.pallas import tpu_sc as plsc

N = 10000
D = 128
ALPHA = 0.1
THETA = 0.5
LAYER = 4
BETA = math.log(THETA / (LAYER + 1) + 1.0)

NC = 2
NS = 16
NW = NC * NS
CHUNK = 128
ROWS_PER_SUB = 624
ROWS_LAST = N - (NS - 1) * ROWS_PER_SUB


def _sc_aggregate(x, ei, zeros_blk):
    E = ei.shape[1]
    assert E % CHUNK == 0
    nchunks = E // CHUNK
    full = nchunks // NW
    rem = nchunks % NW
    assert full % 3 == 0 and full >= 3
    mesh = plsc.VectorSubcoreMesh(core_axis_name="c", subcore_axis_name="s")

    @functools.partial(
        pl.kernel,
        out_type=jax.ShapeDtypeStruct((NC, N, D), jnp.float32),
        mesh=mesh,
        scratch_types=[
            pltpu.VMEM((2, CHUNK), jnp.int32),
            pltpu.VMEM((2, CHUNK), jnp.int32),
            pltpu.VMEM((2, CHUNK), jnp.int32),
            pltpu.VMEM((CHUNK, D), jnp.float32),
            pltpu.VMEM((CHUNK, D), jnp.float32),
            pltpu.VMEM((CHUNK, D), jnp.float32),
            pltpu.VMEM_SHARED((N, D), jnp.float32),
            pltpu.SemaphoreType.DMA,
            pltpu.SemaphoreType.DMA,
            pltpu.SemaphoreType.DMA,
            pltpu.SemaphoreType.DMA,
            pltpu.SemaphoreType.DMA,
            pltpu.SemaphoreType.DMA,
            pltpu.SemaphoreType.DMA,
            pltpu.SemaphoreType.DMA,
            pltpu.SemaphoreType.DMA,
            pltpu.SemaphoreType.DMA,
        ],
    )
    def k(x_hbm, ei_hbm, z_hbm, out_hbm,
          idx0, idx1, idx2, rows0, rows1, rows2, acc,
          isem0, isem1, isem2, gsem0, gsem1, gsem2,
          ssem0, ssem1, ssem2, zsem):
        cid = lax.axis_index("c")
        sid = lax.axis_index("s")
        wid = sid * NC + cid
        tw = full + jnp.where(wid < rem, 1, 0)
        base = pl.multiple_of(sid * ROWS_PER_SUB, 8)
        last = sid == NS - 1

        ibufs = (idx0, idx1, idx2)
        isems = (isem0, isem1, isem2)
        rbufs = (rows0, rows1, rows2)
        gsems = (gsem0, gsem1, gsem2)

        def issue_idx(ch, ibuf, isem):
            off = pl.multiple_of((ch * NW + wid) * CHUNK, CHUNK)
            pltpu.async_copy(ei_hbm.at[0, pl.ds(off, CHUNK)], ibuf.at[0], isem)
            pltpu.async_copy(ei_hbm.at[1, pl.ds(off, CHUNK)], ibuf.at[1], isem)

        def wait_idx(ch, ibuf, isem):
            off = pl.multiple_of((ch * NW + wid) * CHUNK, CHUNK)
            pltpu.make_async_copy(
                ei_hbm.at[0, pl.ds(off, CHUNK)], ibuf.at[0], isem).wait()
            pltpu.make_async_copy(
                ei_hbm.at[1, pl.ds(off, CHUNK)], ibuf.at[1], isem).wait()

        ssems = (ssem0, ssem1, ssem2)

        issue_idx(0, idx0, isem0)
        issue_idx(1, idx1, isem1)
        issue_idx(2, idx2, isem2)
        @pl.when(last)
        def _():
            pltpu.async_copy(z_hbm.at[pl.ds(base, ROWS_LAST)],
                             acc.at[pl.ds(base, ROWS_LAST)], zsem)

        @pl.when(jnp.logical_not(last))
        def _():
            pltpu.async_copy(z_hbm.at[pl.ds(base, ROWS_PER_SUB)],
                             acc.at[pl.ds(base, ROWS_PER_SUB)], zsem)
        wait_idx(0, idx0, isem0)
        pltpu.async_copy(x_hbm.at[idx0.at[0]], rows0, gsem0)
        wait_idx(1, idx1, isem1)
        pltpu.async_copy(x_hbm.at[idx1.at[0]], rows1, gsem1)

        @pl.when(last)
        def _():
            pltpu.make_async_copy(z_hbm.at[pl.ds(base, ROWS_LAST)],
                                  acc.at[pl.ds(base, ROWS_LAST)], zsem).wait()

        @pl.when(jnp.logical_not(last))
        def _():
            pltpu.make_async_copy(z_hbm.at[pl.ds(base, ROWS_PER_SUB)],
                                  acc.at[pl.ds(base, ROWS_PER_SUB)],
                                  zsem).wait()
        plsc.subcore_barrier()

        def body(i, carry):
            for b in (0, 1, 2):
                ch = 3 * i + b
                ibuf, isem = ibufs[b], isems[b]
                rows, gsem, ssem = rbufs[b], gsems[b], ssems[b]
                b2 = (b + 2) % 3
                i2buf, i2sem = ibufs[b2], isems[b2]
                r2buf, g2sem, s2sem = rbufs[b2], gsems[b2], ssems[b2]

                pltpu.make_async_copy(x_hbm.at[ibuf.at[0]], rows, gsem).wait()

                @pl.when(ch + 2 < tw)
                def _():
                    wait_idx(ch + 2, i2buf, i2sem)

                    @pl.when(ch >= 1)
                    def _():
                        pltpu.make_async_copy(
                            r2buf, acc.at[i2buf.at[1]], s2sem).wait()

                    pltpu.async_copy(x_hbm.at[i2buf.at[0]], r2buf, g2sem)

                pltpu.async_copy(rows, acc.at[ibuf.at[1]], ssem, add=True)

                @pl.when(ch + 3 < tw)
                def _():
                    issue_idx(ch + 3, ibuf, isem)
            return carry

        lax.fori_loop(0, full // 3, body, 0)

        @pl.when(tw > full)
        def _():
            pltpu.make_async_copy(x_hbm.at[idx0.at[0]], rows0, gsem0).wait()
            pltpu.async_copy(rows0, acc.at[idx0.at[1]], ssem0, add=True)

        for b in (0, 1, 2):
            pltpu.make_async_copy(
                rbufs[b], acc.at[ibufs[b].at[1]], ssems[b]).wait()

        plsc.subcore_barrier()

        @pl.when(last)
        def _():
            pltpu.sync_copy(acc.at[pl.ds(base, ROWS_LAST)],
                            out_hbm.at[cid, pl.ds(base, ROWS_LAST)])

        @pl.when(jnp.logical_not(last))
        def _():
            pltpu.sync_copy(acc.at[pl.ds(base, ROWS_PER_SUB)],
                            out_hbm.at[cid, pl.ds(base, ROWS_PER_SUB)])

    return k(x, ei, zeros_blk)


_BLK = 5000


def _tc_combine(partial, x_0, W):
    def body(p_ref, x0_ref, w_ref, out_ref):
        t = (1.0 - ALPHA) * (p_ref[0] + p_ref[1]) + ALPHA * x0_ref[...]
        out_ref[...] = (1.0 - BETA) * t + BETA * jnp.dot(
            t, w_ref[...], preferred_element_type=jnp.float32)

    bs = pl.BlockSpec((_BLK, D), lambda i: (i, 0))
    return pl.pallas_call(
        body,
        grid=(N // _BLK,),
        in_specs=[
            pl.BlockSpec((NC, _BLK, D), lambda i: (0, i, 0)),
            bs,
            pl.BlockSpec((D, D), lambda i: (0, 0)),
        ],
        out_specs=bs,
        out_shape=jax.ShapeDtypeStruct((N, D), jnp.float32),
    )(partial, x_0, W)


@jax.jit
def _impl(x, x_0, edge_index, W):
    ei = edge_index.astype(jnp.int32)
    zeros_blk = jnp.zeros((N, D), jnp.float32)
    partial = _sc_aggregate(x, ei, zeros_blk)
    return _tc_combine(partial, x_0, W)


def kernel(x, x_0, edge_index, W):
    return _impl(x, x_0, edge_index, W)
